# baseline (speedup 1.0000x reference)
import jax
import jax.numpy as jnp
from jax import lax
from jax.experimental import pallas as pl
from jax.experimental.pallas import tpu as pltpu
from jax.experimental.pallas import tpu_sc as plsc

N = 10000
D = 128
H = 128
E = 320000
L = 100000

NC = 2
NS = 16
NW = NC * NS

K = 80
EP = 327680
CW = EP // K // NW
SB = 16
NTRASH = 16
NP = 10112
RZ = NP // NS

LP = 100352
PW = LP // NW

_mesh = plsc.VectorSubcoreMesh(core_axis_name="c", subcore_axis_name="s")


def _make_agg_body(with_deg):
    NQ = 8
    NR = 4

    def body(*refs):
        if with_deg:
            (x_hbm, src_hbm, dst_hbm, z_hbm, z1_hbm, ones_hbm,
             agg_out, deg_out) = refs[:8]
            rest = refs[8:]
        else:
            (x_hbm, src_hbm, dst_hbm, z_hbm, agg_out) = refs[:5]
            rest = refs[5:]
        sp = rest[0:NQ]
        dp = rest[NQ:2 * NQ]
        rows = rest[2 * NQ:2 * NQ + NR]
        rest = rest[2 * NQ + NR:]
        if with_deg:
            ones_v, dbuf_v = rest[0], rest[1]
            rest = rest[2:]
        ssem = rest[0:NQ]
        dsem = rest[NQ:2 * NQ]
        gsem = rest[2 * NQ:2 * NQ + NR]
        csem = rest[2 * NQ + NR:2 * NQ + 2 * NR]
        rest = rest[2 * NQ + 2 * NR:]
        agg_sh = rest[0]
        if with_deg:
            deg_sh = rest[1]

        c = lax.axis_index("c")
        s = lax.axis_index("s")
        wid = s * NC + c
        base = wid * CW
        pltpu.sync_copy(z_hbm, agg_sh.at[pl.ds(s * RZ, RZ)])
        if with_deg:
            pltpu.sync_copy(z1_hbm, dbuf_v)
            pltpu.sync_copy(dbuf_v, deg_sh.at[pl.ds(s * RZ, RZ)])
            pltpu.sync_copy(ones_hbm, ones_v)
        plsc.subcore_barrier()

        def idxstart(j, q):
            pltpu.async_copy(src_hbm.at[pl.ds((base + j) * K, K)], sp[q], ssem[q])
            pltpu.async_copy(dst_hbm.at[pl.ds((base + j) * K, K)], dp[q], dsem[q])

        def wait_src(q):
            pltpu.make_async_copy(src_hbm.at[pl.ds(0, K)], sp[q], ssem[q]).wait()

        def wait_dst(q):
            pltpu.make_async_copy(dst_hbm.at[pl.ds(0, K)], dp[q], dsem[q]).wait()

        def gather_start(p, q):
            pltpu.async_copy(x_hbm.at[sp[q]], rows[p], gsem[p])

        def gather_wait(p, q):
            pltpu.make_async_copy(x_hbm.at[sp[q]], rows[p], gsem[p]).wait()

        def scatter_start(p, q):
            pltpu.async_copy(rows[p], agg_sh.at[dp[q]], csem[p], add=True)
            if with_deg:
                pltpu.async_copy(ones_v, deg_sh.at[dp[q]], csem[p], add=True)

        def scatter_wait(p):
            pltpu.make_async_copy(rows[p], agg_sh.at[dp[0]], csem[p]).wait()
            if with_deg:
                pltpu.make_async_copy(ones_v, deg_sh.at[dp[0]], csem[p]).wait()

        def emit_step(jm, n, first):
            p, q = n % NR, n % NQ
            p2, q2 = (n + 2) % NR, (n + 2) % NQ
            qprev = (n - 2) % NQ
            gather_wait(p, q)
            wait_dst(q)
            scatter_start(p, q)
            if first:
                wait_src(q2)
                if n >= 2:
                    scatter_wait(p2)
                    idxstart(jm + NQ - 2, qprev)
                gather_start(p2, q2)
            else:
                @pl.when(jm + 2 < CW)
                def _():
                    wait_src(q2)
                    scatter_wait(p2)
                    gather_start(p2, q2)

                    @pl.when(jm + NQ - 2 < CW)
                    def _():
                        idxstart(jm + NQ - 2, qprev)

        for q in range(NQ):
            idxstart(q, q)
        wait_src(0)
        gather_start(0, 0)
        wait_src(1)
        gather_start(1, 1)
        for n in range(NQ):
            emit_step(n, n, True)

        def body_loop(t, carry):
            j = NQ * t
            for n in range(NQ):
                emit_step(j + n, n, False)
            return carry

        lax.fori_loop(1, CW // NQ, body_loop, 0)
        scatter_wait((CW - 2) % NR)
        scatter_wait((CW - 1) % NR)
        plsc.subcore_barrier()
        pltpu.sync_copy(agg_sh.at[pl.ds(s * RZ, RZ)],
                        agg_out.at[c, pl.ds(s * RZ, RZ)])
        if with_deg:
            pltpu.sync_copy(deg_sh.at[pl.ds(s * RZ, RZ)], dbuf_v)
            pltpu.sync_copy(dbuf_v, deg_out.at[pl.ds(c * NP + s * RZ, RZ)])

    return body


_IDX_SCRATCH = [pltpu.VMEM((K,), jnp.int32) for _ in range(16)]
_ROW_SCRATCH = [pltpu.VMEM((K, D), jnp.float32) for _ in range(4)]
_AGG_DEG = pl.kernel(
    _make_agg_body(True),
    out_type=[
        jax.ShapeDtypeStruct((NC, NP, D), jnp.float32),
        jax.ShapeDtypeStruct((NC * NP,), jnp.float32),
    ],
    mesh=_mesh,
    scratch_types=(
        list(_IDX_SCRATCH)
        + list(_ROW_SCRATCH)
        + [
            pltpu.VMEM((K,), jnp.float32),
            pltpu.VMEM((RZ,), jnp.float32),
        ]
        + [pltpu.SemaphoreType.DMA] * 24
        + [
            pltpu.VMEM_SHARED((NP, D), jnp.float32),
            pltpu.VMEM_SHARED((NP,), jnp.float32),
        ]
    ),
)

_AGG = pl.kernel(
    _make_agg_body(False),
    out_type=[jax.ShapeDtypeStruct((NC, NP, D), jnp.float32)],
    mesh=_mesh,
    scratch_types=(
        list(_IDX_SCRATCH)
        + list(_ROW_SCRATCH)
        + [pltpu.SemaphoreType.DMA] * 24
        + [pltpu.VMEM_SHARED((NP, D), jnp.float32)]
    ),
)


def _final_body(u_hbm, v_hbm, s_hbm, d_hbm, b_hbm,
                out_hbm,
                u_v, v_v, s_v, d_v, o_v, b_v):
    c = lax.axis_index("c")
    s = lax.axis_index("s")
    wid = s * NC + c
    pltpu.sync_copy(u_hbm, u_v)
    pltpu.sync_copy(v_hbm, v_v)
    pltpu.sync_copy(s_hbm.at[pl.ds(wid * PW, PW)], s_v)
    pltpu.sync_copy(d_hbm.at[pl.ds(wid * PW, PW)], d_v)
    pltpu.sync_copy(b_hbm, b_v)
    bb = b_v[...]

    def step(t, carry):
        si = s_v[pl.ds(t * 16, 16)]
        di = d_v[pl.ds(t * 16, 16)]
        us = plsc.load_gather(u_v, [si])
        vd = plsc.load_gather(v_v, [di])
        o_v[pl.ds(t * 16, 16)] = us + vd + bb
        return carry

    lax.fori_loop(0, PW // 16, step, 0)
    pltpu.sync_copy(o_v, out_hbm.at[pl.ds(wid * PW, PW)])


_FINAL = pl.kernel(
    _final_body,
    out_type=[jax.ShapeDtypeStruct((LP,), jnp.float32)],
    mesh=_mesh,
    compiler_params=pltpu.CompilerParams(needs_layout_passes=False),
    scratch_types=[
        pltpu.VMEM((N,), jnp.float32),
        pltpu.VMEM((N,), jnp.float32),
        pltpu.VMEM((PW,), jnp.int32),
        pltpu.VMEM((PW,), jnp.int32),
        pltpu.VMEM((PW,), jnp.float32),
        pltpu.VMEM((16,), jnp.float32),
    ],
)


BR = 1000
G = N // BR


def _make_layer_body(with_uv):
    def body(*refs):
        if with_uv:
            (agg0, agg1, dp0, dp1, x, wl, wr, b, gamma, beta, ws, wd,
             out, u, v, y_s, ps_s, pq_s) = refs
        else:
            (agg0, agg1, dp0, dp1, x, wl, wr, b, gamma, beta,
             out, y_s, ps_s, pq_s) = refs
        i = pl.program_id(0)

        @pl.when(i < G)
        def _():
            deg = jnp.maximum(dp0[...] + dp1[...], 1.0)
            mean = (agg0[0] + agg1[0]) / deg
            yy = lax.dot_general(mean, wl[...], (((1,), (1,)), ((), ())),
                                 preferred_element_type=jnp.float32)
            yy = yy + lax.dot_general(x[...], wr[...], (((1,), (1,)), ((), ())),
                                      preferred_element_type=jnp.float32)
            yy = yy + b[...]
            n2 = jnp.sum(yy * yy, axis=1, keepdims=True)
            yy = yy * lax.rsqrt(jnp.maximum(n2, 1e-24))
            y_s[pl.ds(i, 1)] = yy.reshape(1, BR, H)
            s1 = jnp.sum(yy, axis=0, keepdims=True)
            s2 = jnp.sum(yy * yy, axis=0, keepdims=True)
            ps_s[...] = jnp.where(i == 0, s1, ps_s[...] + s1)
            pq_s[...] = jnp.where(i == 0, s2, pq_s[...] + s2)

        @pl.when(i >= G)
        def _():
            k = i - G
            yy = y_s[pl.ds(k, 1)].reshape(BR, H)
            mu = ps_s[...] * (1.0 / N)
            var = pq_s[...] * (1.0 / N) - mu * mu
            o = (yy - mu) * lax.rsqrt(var + 1e-5) * gamma[...] + beta[...]
            o = jnp.maximum(o, 0.0)
            out[...] = o
            if with_uv:
                u[...] = jnp.sum(o * ws[...], axis=1, keepdims=True)
                v[...] = jnp.sum(o * wd[...], axis=1, keepdims=True)

    return body


def _ph1(i):
    return jnp.where(i < G, i, 0)


def _ph2(i):
    return jnp.where(i >= G, i - G, 0)


_LAYER_IN_SPECS = [
    pl.BlockSpec((1, BR, D), lambda i: (0, _ph1(i), 0)),
    pl.BlockSpec((1, BR, D), lambda i: (1, _ph1(i), 0)),
    pl.BlockSpec((BR, 1), lambda i: (_ph1(i), 0)),
    pl.BlockSpec((BR, 1), lambda i: (_ph1(i), 0)),
    pl.BlockSpec((BR, D), lambda i: (_ph1(i), 0)),
    pl.BlockSpec((H, D), lambda i: (0, 0)),
    pl.BlockSpec((H, D), lambda i: (0, 0)),
    pl.BlockSpec((1, H), lambda i: (0, 0)),
    pl.BlockSpec((1, H), lambda i: (0, 0)),
    pl.BlockSpec((1, H), lambda i: (0, 0)),
]

_LAYER_SCRATCH = [
    pltpu.VMEM((G, BR, H), jnp.float32),
    pltpu.VMEM((1, H), jnp.float32),
    pltpu.VMEM((1, H), jnp.float32),
]

_LAYER1 = pl.pallas_call(
    _make_layer_body(False),
    grid=(2 * G,),
    in_specs=list(_LAYER_IN_SPECS),
    out_specs=pl.BlockSpec((BR, H), lambda i: (_ph2(i), 0)),
    out_shape=jax.ShapeDtypeStruct((N, H), jnp.float32),
    scratch_shapes=list(_LAYER_SCRATCH),
)

_LAYER2 = pl.pallas_call(
    _make_layer_body(True),
    grid=(2 * G,),
    in_specs=list(_LAYER_IN_SPECS) + [
        pl.BlockSpec((1, H), lambda i: (0, 0)),
        pl.BlockSpec((1, H), lambda i: (0, 0)),
    ],
    out_specs=[
        pl.BlockSpec((BR, H), lambda i: (_ph2(i), 0)),
        pl.BlockSpec((BR, 1), lambda i: (_ph2(i), 0)),
        pl.BlockSpec((BR, 1), lambda i: (_ph2(i), 0)),
    ],
    out_shape=[
        jax.ShapeDtypeStruct((N, H), jnp.float32),
        jax.ShapeDtypeStruct((N, 1), jnp.float32),
        jax.ShapeDtypeStruct((N, 1), jnp.float32),
    ],
    scratch_shapes=list(_LAYER_SCRATCH),
)


@jax.jit
def _forward(edge_index, edge_label_index, embedding, W_l1, W_r1, b1,
             gamma1, beta1, W_l2, W_r2, b2, gamma2, beta2, W_lin, b_lin):
    src = edge_index[0]
    dst = edge_index[1]
    ar = jnp.arange(EP - E, dtype=jnp.int32)
    src1 = jnp.concatenate([src, ar % N])
    dst1 = jnp.concatenate([dst, N + (ar % NTRASH)])
    zeros = jnp.zeros((RZ, D), jnp.float32)
    zeros1 = jnp.zeros((RZ,), jnp.float32)
    ones1 = jnp.ones((K,), jnp.float32)

    aggp1, degp = _AGG_DEG(embedding, src1, dst1, zeros, zeros1, ones1)
    dp0 = degp[:N].reshape(N, 1)
    dp1 = degp[NP:NP + N].reshape(N, 1)
    x1 = _LAYER1(aggp1, aggp1, dp0, dp1, embedding, W_l1, W_r1,
                 b1.reshape(1, H), gamma1.reshape(1, H), beta1.reshape(1, H))

    aggp2 = _AGG(x1, src1, dst1, zeros)
    if isinstance(aggp2, (list, tuple)):
        aggp2 = aggp2[0]
    x2, u, v = _LAYER2(aggp2, aggp2, dp0, dp1, x1, W_l2, W_r2,
                       b2.reshape(1, H), gamma2.reshape(1, H),
                       beta2.reshape(1, H), W_lin[:, :H], W_lin[:, H:])
    del x2

    sl = edge_label_index[0]
    dl = edge_label_index[1]
    zpad = jnp.zeros((LP - L,), jnp.int32)
    sp = jnp.concatenate([sl, zpad])
    dp = jnp.concatenate([dl, zpad])
    b16 = jnp.broadcast_to(b_lin.astype(jnp.float32), (16,))
    outp = _FINAL(u.reshape(-1), v.reshape(-1), sp, dp, b16)
    if isinstance(outp, (list, tuple)):
        outp = outp[0]
    return outp[:L]


def kernel(edge_index, edge_label_index, embedding, W_l1, W_r1, b1, gamma1,
           beta1, W_l2, W_r2, b2, gamma2, beta2, W_lin, b_lin):
    return _forward(edge_index, edge_label_index, embedding, W_l1, W_r1, b1,
                    gamma1, beta1, W_l2, W_r2, b2, gamma2, beta2, W_lin, b_lin)


# speedup vs baseline: 1.0741x; 1.0741x over previous
import jax
import jax.numpy as jnp
from jax import lax
from jax.experimental import pallas as pl
from jax.experimental.pallas import tpu as pltpu
from jax.experimental.pallas import tpu_sc as plsc

N = 10000
D = 128
H = 128
E = 320000
L = 100000

NC = 2
NS = 16
NW = NC * NS

K = 128
EP = 327680
CW = EP // K // NW
SB = 16
NTRASH = 16
NP = 10112
RZ = NP // NS

LP = 100352
PW = LP // NW

_mesh = plsc.VectorSubcoreMesh(core_axis_name="c", subcore_axis_name="s")


def _make_agg_body(with_deg):
    def body(*refs):
        if with_deg:
            (x_hbm, src_hbm, dst_hbm, z_hbm, z1_hbm, ones_hbm,
             agg_out, deg_out,
             s0, d0, s1, d1, s2, d2, s3, d3, rowsA, rowsB, ones_v, dbuf_v,
             ss0, sd0, ss1, sd1, ss2, sd2, ss3, sd3, gsA, gsB,
             agg_sh, deg_sh) = refs
        else:
            (x_hbm, src_hbm, dst_hbm, z_hbm,
             agg_out,
             s0, d0, s1, d1, s2, d2, s3, d3, rowsA, rowsB,
             ss0, sd0, ss1, sd1, ss2, sd2, ss3, sd3, gsA, gsB,
             agg_sh) = refs
        sp = [s0, s1, s2, s3]
        dp = [d0, d1, d2, d3]
        ssem = [ss0, ss1, ss2, ss3]
        dsem = [sd0, sd1, sd2, sd3]
        c = lax.axis_index("c")
        s = lax.axis_index("s")
        wid = s * NC + c
        base = wid * CW
        pltpu.sync_copy(z_hbm, agg_sh.at[pl.ds(s * RZ, RZ)])
        if with_deg:
            pltpu.sync_copy(z1_hbm, dbuf_v)
            pltpu.sync_copy(dbuf_v, deg_sh.at[pl.ds(s * RZ, RZ)])
            pltpu.sync_copy(ones_hbm, ones_v)
        plsc.subcore_barrier()

        def idxstart(j, p):
            pltpu.async_copy(src_hbm.at[pl.ds((base + j) * K, K)], sp[p], ssem[p])
            pltpu.async_copy(dst_hbm.at[pl.ds((base + j) * K, K)], dp[p], dsem[p])

        def wait_src(p):
            pltpu.make_async_copy(src_hbm.at[pl.ds(0, K)], sp[p], ssem[p]).wait()

        def wait_dst(p):
            pltpu.make_async_copy(dst_hbm.at[pl.ds(0, K)], dp[p], dsem[p]).wait()

        def gather_start(rows, p, gs):
            pltpu.async_copy(x_hbm.at[sp[p]], rows, gs)

        def gather_wait(rows, p, gs):
            pltpu.make_async_copy(x_hbm.at[sp[p]], rows, gs).wait()

        def scatter(rows, p):
            pltpu.sync_copy(rows, agg_sh.at[dp[p]], add=True)
            if with_deg:
                pltpu.sync_copy(ones_v, deg_sh.at[dp[p]], add=True)

        for p in range(4):
            idxstart(p, p)
        wait_src(0)
        gather_start(rowsA, 0, gsA)

        def body_loop(t, carry):
            j = 4 * t
            wait_src(1)
            gather_start(rowsB, 1, gsB)
            gather_wait(rowsA, 0, gsA)
            wait_dst(0)
            scatter(rowsA, 0)

            @pl.when(j + 4 < CW)
            def _():
                idxstart(j + 4, 0)

            wait_src(2)
            gather_start(rowsA, 2, gsA)
            gather_wait(rowsB, 1, gsB)
            wait_dst(1)
            scatter(rowsB, 1)

            @pl.when(j + 5 < CW)
            def _():
                idxstart(j + 5, 1)

            wait_src(3)
            gather_start(rowsB, 3, gsB)
            gather_wait(rowsA, 2, gsA)
            wait_dst(2)
            scatter(rowsA, 2)

            @pl.when(j + 6 < CW)
            def _():
                idxstart(j + 6, 2)

            @pl.when(j + 4 < CW)
            def _():
                wait_src(0)
                gather_start(rowsA, 0, gsA)

            gather_wait(rowsB, 3, gsB)
            wait_dst(3)
            scatter(rowsB, 3)

            @pl.when(j + 7 < CW)
            def _():
                idxstart(j + 7, 3)

            return carry

        lax.fori_loop(0, CW // 4, body_loop, 0)
        plsc.subcore_barrier()
        pltpu.sync_copy(agg_sh.at[pl.ds(s * RZ, RZ)],
                        agg_out.at[c, pl.ds(s * RZ, RZ)])
        if with_deg:
            pltpu.sync_copy(deg_sh.at[pl.ds(s * RZ, RZ)], dbuf_v)
            pltpu.sync_copy(dbuf_v, deg_out.at[pl.ds(c * NP + s * RZ, RZ)])

    return body


_IDX_SCRATCH = [pltpu.VMEM((K,), jnp.int32) for _ in range(8)]
_AGG_DEG = pl.kernel(
    _make_agg_body(True),
    out_type=[
        jax.ShapeDtypeStruct((NC, NP, D), jnp.float32),
        jax.ShapeDtypeStruct((NC * NP,), jnp.float32),
    ],
    mesh=_mesh,
    scratch_types=(
        list(_IDX_SCRATCH)
        + [
            pltpu.VMEM((K, D), jnp.float32),
            pltpu.VMEM((K, D), jnp.float32),
            pltpu.VMEM((K,), jnp.float32),
            pltpu.VMEM((RZ,), jnp.float32),
        ]
        + [pltpu.SemaphoreType.DMA] * 10
        + [
            pltpu.VMEM_SHARED((NP, D), jnp.float32),
            pltpu.VMEM_SHARED((NP,), jnp.float32),
        ]
    ),
)

_AGG = pl.kernel(
    _make_agg_body(False),
    out_type=[jax.ShapeDtypeStruct((NC, NP, D), jnp.float32)],
    mesh=_mesh,
    scratch_types=(
        list(_IDX_SCRATCH)
        + [
            pltpu.VMEM((K, D), jnp.float32),
            pltpu.VMEM((K, D), jnp.float32),
        ]
        + [pltpu.SemaphoreType.DMA] * 10
        + [pltpu.VMEM_SHARED((NP, D), jnp.float32)]
    ),
)


def _final_body(u_hbm, v_hbm, s_hbm, d_hbm, b_hbm,
                out_hbm,
                u_v, v_v, s_v, d_v, o_v, b_v):
    c = lax.axis_index("c")
    s = lax.axis_index("s")
    wid = s * NC + c
    pltpu.sync_copy(u_hbm, u_v)
    pltpu.sync_copy(v_hbm, v_v)
    pltpu.sync_copy(s_hbm.at[pl.ds(wid * PW, PW)], s_v)
    pltpu.sync_copy(d_hbm.at[pl.ds(wid * PW, PW)], d_v)
    pltpu.sync_copy(b_hbm, b_v)
    bb = b_v[...]

    def step(t, carry):
        si = s_v[pl.ds(t * 16, 16)]
        di = d_v[pl.ds(t * 16, 16)]
        us = plsc.load_gather(u_v, [si])
        vd = plsc.load_gather(v_v, [di])
        o_v[pl.ds(t * 16, 16)] = us + vd + bb
        return carry

    lax.fori_loop(0, PW // 16, step, 0)
    pltpu.sync_copy(o_v, out_hbm.at[pl.ds(wid * PW, PW)])


_FINAL = pl.kernel(
    _final_body,
    out_type=[jax.ShapeDtypeStruct((LP,), jnp.float32)],
    mesh=_mesh,
    compiler_params=pltpu.CompilerParams(needs_layout_passes=False),
    scratch_types=[
        pltpu.VMEM((N,), jnp.float32),
        pltpu.VMEM((N,), jnp.float32),
        pltpu.VMEM((PW,), jnp.int32),
        pltpu.VMEM((PW,), jnp.int32),
        pltpu.VMEM((PW,), jnp.float32),
        pltpu.VMEM((16,), jnp.float32),
    ],
)


BR = 2000
G = N // BR


def _make_layer_body(with_uv):
    def body(*refs):
        if with_uv:
            (agg0, agg1, dp0, dp1, x, wl, wr, b, gamma, beta, ws, wd,
             out, u, v, y_s, ps_s, pq_s) = refs
        else:
            (agg0, agg1, dp0, dp1, x, wl, wr, b, gamma, beta,
             out, y_s, ps_s, pq_s) = refs
        i = pl.program_id(0)

        @pl.when(i < G)
        def _():
            deg = jnp.maximum(dp0[...] + dp1[...], 1.0)
            mean = (agg0[0] + agg1[0]) / deg
            yy = lax.dot_general(mean, wl[...], (((1,), (1,)), ((), ())),
                                 preferred_element_type=jnp.float32)
            yy = yy + lax.dot_general(x[...], wr[...], (((1,), (1,)), ((), ())),
                                      preferred_element_type=jnp.float32)
            yy = yy + b[...]
            n2 = jnp.sum(yy * yy, axis=1, keepdims=True)
            yy = yy * lax.rsqrt(jnp.maximum(n2, 1e-24))
            y_s[pl.ds(i, 1)] = yy.reshape(1, BR, H)
            s1 = jnp.sum(yy, axis=0, keepdims=True)
            s2 = jnp.sum(yy * yy, axis=0, keepdims=True)
            ps_s[...] = jnp.where(i == 0, s1, ps_s[...] + s1)
            pq_s[...] = jnp.where(i == 0, s2, pq_s[...] + s2)

        @pl.when(i >= G)
        def _():
            k = i - G
            yy = y_s[pl.ds(k, 1)].reshape(BR, H)
            mu = ps_s[...] * (1.0 / N)
            var = pq_s[...] * (1.0 / N) - mu * mu
            o = (yy - mu) * lax.rsqrt(var + 1e-5) * gamma[...] + beta[...]
            o = jnp.maximum(o, 0.0)
            out[...] = o
            if with_uv:
                u[...] = jnp.sum(o * ws[...], axis=1, keepdims=True)
                v[...] = jnp.sum(o * wd[...], axis=1, keepdims=True)

    return body


def _ph1(i):
    return jnp.where(i < G, i, 0)


def _ph2(i):
    return jnp.where(i >= G, i - G, 0)


_LAYER_IN_SPECS = [
    pl.BlockSpec((1, BR, D), lambda i: (0, _ph1(i), 0)),
    pl.BlockSpec((1, BR, D), lambda i: (1, _ph1(i), 0)),
    pl.BlockSpec((BR, 1), lambda i: (_ph1(i), 0)),
    pl.BlockSpec((BR, 1), lambda i: (_ph1(i), 0)),
    pl.BlockSpec((BR, D), lambda i: (_ph1(i), 0)),
    pl.BlockSpec((H, D), lambda i: (0, 0)),
    pl.BlockSpec((H, D), lambda i: (0, 0)),
    pl.BlockSpec((1, H), lambda i: (0, 0)),
    pl.BlockSpec((1, H), lambda i: (0, 0)),
    pl.BlockSpec((1, H), lambda i: (0, 0)),
]

_LAYER_SCRATCH = [
    pltpu.VMEM((G, BR, H), jnp.float32),
    pltpu.VMEM((1, H), jnp.float32),
    pltpu.VMEM((1, H), jnp.float32),
]

_LAYER1 = pl.pallas_call(
    _make_layer_body(False),
    grid=(2 * G,),
    in_specs=list(_LAYER_IN_SPECS),
    out_specs=pl.BlockSpec((BR, H), lambda i: (_ph2(i), 0)),
    out_shape=jax.ShapeDtypeStruct((N, H), jnp.float32),
    scratch_shapes=list(_LAYER_SCRATCH),
)

_LAYER2 = pl.pallas_call(
    _make_layer_body(True),
    grid=(2 * G,),
    in_specs=list(_LAYER_IN_SPECS) + [
        pl.BlockSpec((1, H), lambda i: (0, 0)),
        pl.BlockSpec((1, H), lambda i: (0, 0)),
    ],
    out_specs=[
        pl.BlockSpec((BR, H), lambda i: (_ph2(i), 0)),
        pl.BlockSpec((BR, 1), lambda i: (_ph2(i), 0)),
        pl.BlockSpec((BR, 1), lambda i: (_ph2(i), 0)),
    ],
    out_shape=[
        jax.ShapeDtypeStruct((N, H), jnp.float32),
        jax.ShapeDtypeStruct((N, 1), jnp.float32),
        jax.ShapeDtypeStruct((N, 1), jnp.float32),
    ],
    scratch_shapes=list(_LAYER_SCRATCH),
)


@jax.jit
def _forward(edge_index, edge_label_index, embedding, W_l1, W_r1, b1,
             gamma1, beta1, W_l2, W_r2, b2, gamma2, beta2, W_lin, b_lin):
    src = edge_index[0]
    dst = edge_index[1]
    ar = jnp.arange(EP - E, dtype=jnp.int32)
    src1 = jnp.concatenate([src, ar % N])
    dst1 = jnp.concatenate([dst, N + (ar % NTRASH)])
    zeros = jnp.zeros((RZ, D), jnp.float32)
    zeros1 = jnp.zeros((RZ,), jnp.float32)
    ones1 = jnp.ones((K,), jnp.float32)

    aggp1, degp = _AGG_DEG(embedding, src1, dst1, zeros, zeros1, ones1)
    dp0 = degp[:N].reshape(N, 1)
    dp1 = degp[NP:NP + N].reshape(N, 1)
    x1 = _LAYER1(aggp1, aggp1, dp0, dp1, embedding, W_l1, W_r1,
                 b1.reshape(1, H), gamma1.reshape(1, H), beta1.reshape(1, H))

    aggp2 = _AGG(x1, src1, dst1, zeros)
    if isinstance(aggp2, (list, tuple)):
        aggp2 = aggp2[0]
    x2, u, v = _LAYER2(aggp2, aggp2, dp0, dp1, x1, W_l2, W_r2,
                       b2.reshape(1, H), gamma2.reshape(1, H),
                       beta2.reshape(1, H), W_lin[:, :H], W_lin[:, H:])
    del x2

    sl = edge_label_index[0]
    dl = edge_label_index[1]
    zpad = jnp.zeros((LP - L,), jnp.int32)
    sp = jnp.concatenate([sl, zpad])
    dp = jnp.concatenate([dl, zpad])
    b16 = jnp.broadcast_to(b_lin.astype(jnp.float32), (16,))
    outp = _FINAL(u.reshape(-1), v.reshape(-1), sp, dp, b16)
    if isinstance(outp, (list, tuple)):
        outp = outp[0]
    return outp[:L]


def kernel(edge_index, edge_label_index, embedding, W_l1, W_r1, b1, gamma1,
           beta1, W_l2, W_r2, b2, gamma2, beta2, W_lin, b_lin):
    return _forward(edge_index, edge_label_index, embedding, W_l1, W_r1, b1,
                    gamma1, beta1, W_l2, W_r2, b2, gamma2, beta2, W_lin, b_lin)
